# baseline (speedup 1.0000x reference)
import functools

import jax
import jax.numpy as jnp
from jax import lax
from jax.experimental import pallas as pl
from jax.experimental.pallas import tpu as pltpu
from jax.experimental.pallas import tpu_sc as plsc

_LANES = 16
_CHUNK = 128
_MAXLEN = 2048


@functools.lru_cache(maxsize=None)
def _padder(nbatch, maxlen, total, d):
    info = plsc.get_sparse_core_info()
    nc, ns = info.num_cores, info.num_subcores
    nw = nc * ns
    out_rows = nbatch * maxlen
    assert out_rows % nw == 0
    rpw = out_rows // nw
    assert maxlen % rpw == 0 and rpw % _CHUNK == 0
    wpb = maxlen // rpw
    nchunks = rpw // _CHUNK
    mesh = plsc.VectorSubcoreMesh(core_axis_name="c", subcore_axis_name="s")

    @functools.partial(
        pl.kernel,
        mesh=mesh,
        out_type=jax.ShapeDtypeStruct((nbatch, maxlen, d), jnp.float32),
        scratch_types=[
            pltpu.VMEM((_LANES,), jnp.int32),
            pltpu.VMEM((nchunks, _CHUNK), jnp.int32),
            pltpu.VMEM((2, _CHUNK, d), jnp.float32),
            pltpu.SemaphoreType.DMA,
            pltpu.SemaphoreType.DMA,
        ],
    )
    def k(flat_hbm, cu_hbm, out_hbm, cu_v, idx_v, rows_v, sem0, sem1):
        wid = lax.axis_index("s") * nc + lax.axis_index("c")
        b = wid // wpb
        t0 = (wid % wpb) * rpw
        pltpu.sync_copy(cu_hbm, cu_v)
        cu_vec = cu_v[...]
        iota = lax.iota(jnp.int32, _LANES)

        gd = lax.GatherDimensionNumbers(
            offset_dims=(), collapsed_slice_dims=(0,), start_index_map=(0,)
        )

        def lane_bcast(i):
            idxs = jnp.broadcast_to(jnp.asarray(i, jnp.int32), (_LANES, 1))
            return lax.gather(
                cu_vec, idxs, gd, slice_sizes=(1,),
                mode=lax.GatherScatterMode.PROMISE_IN_BOUNDS,
            )

        cu_b = lane_bcast(b)
        cu_b1 = lane_bcast(b + 1)
        ml = lane_bcast(_LANES - 1)
        length = jnp.minimum(cu_b1 - cu_b, ml)
        for j in range(rpw // _LANES):
            t = t0 + j * _LANES + iota
            idx = jnp.where(t < length, cu_b + t, total)
            idx_v[j * _LANES // _CHUNK, pl.ds((j * _LANES) % _CHUNK, _LANES)] = idx

        sems = (sem0, sem1)

        def gather(c):
            return pltpu.async_copy(
                flat_hbm.at[idx_v.at[c]], rows_v.at[c % 2], sems[c % 2]
            )

        h = gather(0)
        for c in range(nchunks):
            h.wait()
            if c + 1 < nchunks:
                h = gather(c + 1)
            pltpu.sync_copy(
                rows_v.at[c % 2], out_hbm.at[b, pl.ds(t0 + c * _CHUNK, _CHUNK)]
            )

    return k


def kernel(flat, cu_seqlens, maxlen):
    total, d = flat.shape
    nbatch = cu_seqlens.shape[0] - 1
    flat_ext = jnp.concatenate([flat, jnp.zeros((8, d), flat.dtype)], axis=0)
    cu_pad = (
        jnp.zeros((_LANES,), jnp.int32)
        .at[: nbatch + 1]
        .set(cu_seqlens)
        .at[_LANES - 1]
        .set(jnp.asarray(maxlen, jnp.int32))
    )
    padded = _padder(nbatch, _MAXLEN, total, d)(flat_ext, cu_pad)
    lengths = cu_seqlens[1:] - cu_seqlens[:-1]
    return padded, lengths


# speedup vs baseline: 1.0008x; 1.0008x over previous
import functools

import jax
import jax.numpy as jnp
from jax import lax
from jax.experimental import pallas as pl
from jax.experimental.pallas import tpu as pltpu
from jax.experimental.pallas import tpu_sc as plsc

_LANES = 16
_CHUNK = 128
_MAXLEN = 2048


@functools.lru_cache(maxsize=None)
def _padder(nbatch, maxlen, total, d):
    info = plsc.get_sparse_core_info()
    nc, ns = info.num_cores, info.num_subcores
    nw = nc * ns
    out_rows = nbatch * maxlen
    assert out_rows % nw == 0
    rpw = out_rows // nw
    assert maxlen % rpw == 0 and rpw % _CHUNK == 0
    wpb = maxlen // rpw
    nchunks = rpw // _CHUNK
    mesh = plsc.VectorSubcoreMesh(core_axis_name="c", subcore_axis_name="s")

    @functools.partial(
        pl.kernel,
        mesh=mesh,
        out_type=jax.ShapeDtypeStruct((nbatch, maxlen, d), jnp.float32),
        scratch_types=[
            pltpu.VMEM((_LANES,), jnp.int32),
            pltpu.VMEM((nchunks, _CHUNK), jnp.int32),
            pltpu.VMEM((nchunks, _CHUNK, d), jnp.float32),
            pltpu.SemaphoreType.DMA((nchunks,)),
            pltpu.SemaphoreType.DMA,
        ],
    )
    def k(flat_hbm, cu_hbm, out_hbm, cu_v, idx_v, rows_v, gsem, osem):
        wid = lax.axis_index("s") * nc + lax.axis_index("c")
        b = wid // wpb
        t0 = (wid % wpb) * rpw
        pltpu.sync_copy(cu_hbm, cu_v)
        cu_vec = cu_v[...]
        iota = lax.iota(jnp.int32, _LANES)

        gd = lax.GatherDimensionNumbers(
            offset_dims=(), collapsed_slice_dims=(0,), start_index_map=(0,)
        )

        def lane_bcast(i):
            idxs = jnp.broadcast_to(jnp.asarray(i, jnp.int32), (_LANES, 1))
            return lax.gather(
                cu_vec, idxs, gd, slice_sizes=(1,),
                mode=lax.GatherScatterMode.PROMISE_IN_BOUNDS,
            )

        cu_b = lane_bcast(b)
        cu_b1 = lane_bcast(b + 1)
        ml = lane_bcast(_LANES - 1)
        length = jnp.minimum(cu_b1 - cu_b, ml)
        for j in range(rpw // _LANES):
            t = t0 + j * _LANES + iota
            idx = jnp.where(t < length, cu_b + t, total)
            idx_v[j * _LANES // _CHUNK, pl.ds((j * _LANES) % _CHUNK, _LANES)] = idx

        gathers = [
            pltpu.async_copy(flat_hbm.at[idx_v.at[c]], rows_v.at[c], gsem.at[c])
            for c in range(nchunks)
        ]
        outs = []
        for c in range(nchunks):
            gathers[c].wait()
            outs.append(
                pltpu.async_copy(
                    rows_v.at[c], out_hbm.at[b, pl.ds(t0 + c * _CHUNK, _CHUNK)], osem
                )
            )
        for o in outs:
            o.wait()

    return k


def kernel(flat, cu_seqlens, maxlen):
    total, d = flat.shape
    nbatch = cu_seqlens.shape[0] - 1
    flat_ext = jnp.concatenate([flat, jnp.zeros((8, d), flat.dtype)], axis=0)
    cu_pad = (
        jnp.zeros((_LANES,), jnp.int32)
        .at[: nbatch + 1]
        .set(cu_seqlens)
        .at[_LANES - 1]
        .set(jnp.asarray(maxlen, jnp.int32))
    )
    padded = _padder(nbatch, _MAXLEN, total, d)(flat_ext, cu_pad)
    lengths = cu_seqlens[1:] - cu_seqlens[:-1]
    return padded, lengths


# speedup vs baseline: 11.5717x; 11.5625x over previous
import functools

import jax
import jax.numpy as jnp
from jax import lax
from jax.experimental import pallas as pl
from jax.experimental.pallas import tpu as pltpu
from jax.experimental.pallas import tpu_sc as plsc

_LANES = 16
_CHUNK = 128
_MAXLEN = 2048


@functools.lru_cache(maxsize=None)
def _padder(nbatch, maxlen, total, d):
    info = plsc.get_sparse_core_info()
    nc, ns = info.num_cores, info.num_subcores
    nw = nc * ns
    out_rows = nbatch * maxlen
    assert out_rows % nw == 0
    rpw = out_rows // nw
    assert maxlen % rpw == 0 and rpw % _CHUNK == 0
    wpb = maxlen // rpw
    nchunks = rpw // _CHUNK
    mesh = plsc.VectorSubcoreMesh(core_axis_name="c", subcore_axis_name="s")

    @functools.partial(
        pl.kernel,
        mesh=mesh,
        out_type=jax.ShapeDtypeStruct((nbatch, maxlen, d), jnp.float32),
        compiler_params=pltpu.CompilerParams(needs_layout_passes=False),
        scratch_types=[
            pltpu.VMEM((_LANES,), jnp.int32),
            pltpu.VMEM((_CHUNK,), jnp.int32),
            pltpu.VMEM((nchunks, _CHUNK, d), jnp.float32),
            pltpu.VMEM((_CHUNK, d), jnp.float32),
            pltpu.SemaphoreType.DMA((nchunks,)),
            pltpu.SemaphoreType.DMA,
            pltpu.SemaphoreType.DMA,
        ],
    )
    def k(flat_hbm, cu_hbm, zero_hbm, out_hbm,
          cu_v, idx_v, rows_v, zbuf, isem, zsem, osem):
        wid = lax.axis_index("s") * nc + lax.axis_index("c")
        b = wid // wpb
        t0 = pl.multiple_of((wid % wpb) * rpw, _CHUNK)
        zstage = pltpu.async_copy(zero_hbm, zbuf, zsem)
        pltpu.sync_copy(cu_hbm, cu_v)
        cu_vec = cu_v[...]
        iota = lax.iota(jnp.int32, _LANES)

        def lane_scalar(i):
            return jnp.sum(jnp.where(iota == i, cu_vec, 0))

        start = lane_scalar(b)
        length = jnp.minimum(lane_scalar(b + 1) - start, lane_scalar(_LANES - 1))
        nv = jnp.clip(length - t0, 0, rpw)
        aligned = start % 8 == 0

        for c in range(nchunks):
            c0, c1 = c * _CHUNK, (c + 1) * _CHUNK
            fast = (nv >= c1) & aligned

            @pl.when(fast)
            def _full():
                src = pl.multiple_of((start // 8) * 8 + t0 + c0, 8)
                pltpu.async_copy(
                    flat_hbm.at[pl.ds(src, _CHUNK)], rows_v.at[c], isem.at[c]
                )

            @pl.when(~fast & (nv > c0))
            def _gather():
                for j in range(_CHUNK // _LANES):
                    t = t0 + c0 + j * _LANES + iota
                    idx = jnp.where(t < length, start + t, total)
                    idx_v[pl.ds(j * _LANES, _LANES)] = idx
                pltpu.async_copy(flat_hbm.at[idx_v], rows_v.at[c], isem.at[c])

        zstage.wait()

        for c in range(nchunks):
            c0, c1 = c * _CHUNK, (c + 1) * _CHUNK
            dst = out_hbm.at[b, pl.ds(t0 + c0, _CHUNK)]

            @pl.when(nv > c0)
            def _data():
                pltpu.make_async_copy(
                    flat_hbm.at[pl.ds(0, _CHUNK)], rows_v.at[c], isem.at[c]
                ).wait()
                pltpu.async_copy(rows_v.at[c], dst, osem)

            @pl.when(nv <= c0)
            def _pad():
                pltpu.async_copy(zbuf, dst, osem)

        for c in range(nchunks):
            pltpu.make_async_copy(zero_hbm, zbuf, osem).wait()

    return k


def kernel(flat, cu_seqlens, maxlen):
    total, d = flat.shape
    nbatch = cu_seqlens.shape[0] - 1
    flat_ext = jnp.concatenate([flat, jnp.zeros((_CHUNK + 8, d), flat.dtype)], 0)
    zero_block = jnp.zeros((_CHUNK, d), flat.dtype)
    cu_pad = (
        jnp.zeros((_LANES,), jnp.int32)
        .at[: nbatch + 1]
        .set(cu_seqlens)
        .at[_LANES - 1]
        .set(jnp.asarray(maxlen, jnp.int32))
    )
    padded = _padder(nbatch, _MAXLEN, total, d)(flat_ext, cu_pad, zero_block)
    lengths = cu_seqlens[1:] - cu_seqlens[:-1]
    return padded, lengths


# speedup vs baseline: 11.8955x; 1.0280x over previous
import functools

import jax
import jax.numpy as jnp
from jax import lax
from jax.experimental import pallas as pl
from jax.experimental.pallas import tpu as pltpu
from jax.experimental.pallas import tpu_sc as plsc

_LANES = 16
_CHUNK = 128
_MAXLEN = 2048


@functools.lru_cache(maxsize=None)
def _padder(nbatch, maxlen, total, d):
    info = plsc.get_sparse_core_info()
    nc, ns = info.num_cores, info.num_subcores
    nw = nc * ns
    out_rows = nbatch * maxlen
    assert out_rows % nw == 0
    rpw = out_rows // nw
    assert maxlen % rpw == 0 and rpw % _CHUNK == 0
    wpb = maxlen // rpw
    nchunks = rpw // _CHUNK
    mesh = plsc.VectorSubcoreMesh(core_axis_name="c", subcore_axis_name="s")

    @functools.partial(
        pl.kernel,
        mesh=mesh,
        out_type=jax.ShapeDtypeStruct((nbatch, maxlen, d), jnp.float32),
        compiler_params=pltpu.CompilerParams(needs_layout_passes=False),
        scratch_types=[
            pltpu.VMEM((_LANES,), jnp.int32),
            pltpu.VMEM((_CHUNK,), jnp.int32),
            pltpu.VMEM((nchunks, _CHUNK, d), jnp.float32),
            pltpu.VMEM((_CHUNK, d), jnp.float32),
            pltpu.SemaphoreType.DMA((nchunks,)),
            pltpu.SemaphoreType.DMA,
            pltpu.SemaphoreType.DMA,
        ],
    )
    def k(flat_hbm, cu_hbm, zero_hbm, out_hbm,
          cu_v, idx_v, rows_v, zbuf, isem, zsem, osem):
        wid = lax.axis_index("s") * nc + lax.axis_index("c")
        b = wid // wpb
        t0 = pl.multiple_of((wid % wpb) * rpw, _CHUNK)
        zstage = pltpu.async_copy(zero_hbm, zbuf, zsem)
        pltpu.sync_copy(cu_hbm, cu_v)
        cu_vec = cu_v[...]
        iota = lax.iota(jnp.int32, _LANES)

        def lane_scalar(i):
            return jnp.sum(jnp.where(iota == i, cu_vec, 0))

        start = lane_scalar(b)
        length = jnp.minimum(lane_scalar(b + 1) - start, lane_scalar(_LANES - 1))
        nv = jnp.clip(length - t0, 0, rpw)
        aligned = start % 8 == 0
        fast_cs = [(nv >= (c + 1) * _CHUNK) & aligned for c in range(nchunks)]

        for c in range(nchunks):
            c0 = c * _CHUNK

            @pl.when(fast_cs[c])
            def _full():
                src = pl.multiple_of((start // 8) * 8 + t0 + c0, 8)
                pltpu.async_copy(
                    flat_hbm.at[pl.ds(src, _CHUNK)], rows_v.at[c], isem.at[c]
                )

            @pl.when(~fast_cs[c] & (nv > c0))
            def _gather():
                for j in range(_CHUNK // _LANES):
                    t = t0 + c0 + j * _LANES + iota
                    idx = jnp.clip(jnp.where(t < length, start + t, 0), 0, total - 1)
                    idx_v[pl.ds(j * _LANES, _LANES)] = idx
                pltpu.async_copy(flat_hbm.at[idx_v], rows_v.at[c], isem.at[c])

        zstage.wait()

        for c in range(nchunks):
            c0 = c * _CHUNK
            dst = out_hbm.at[b, pl.ds(t0 + c0, _CHUNK)]

            @pl.when(nv > c0)
            def _wait_in():
                pltpu.make_async_copy(
                    flat_hbm.at[pl.ds(0, _CHUNK)], rows_v.at[c], isem.at[c]
                ).wait()

            @pl.when(~fast_cs[c] & (nv > c0))
            def _mask_tail():
                def body(row, carry):
                    for g in range(d // _LANES):
                        rows_v[c, row, pl.ds(g * _LANES, _LANES)] = jnp.zeros(
                            (_LANES,), jnp.float32
                        )
                    return carry

                lax.fori_loop(jnp.maximum(nv - c0, 0), _CHUNK, body, 0)

            @pl.when(nv > c0)
            def _data():
                pltpu.async_copy(rows_v.at[c], dst, osem)

            @pl.when(nv <= c0)
            def _pad():
                pltpu.async_copy(zbuf, dst, osem)

        for c in range(nchunks):
            pltpu.make_async_copy(zero_hbm, zbuf, osem).wait()

    return k


def kernel(flat, cu_seqlens, maxlen):
    total, d = flat.shape
    nbatch = cu_seqlens.shape[0] - 1
    zero_block = jnp.zeros((_CHUNK, d), flat.dtype)
    cu_pad = (
        jnp.zeros((_LANES,), jnp.int32)
        .at[: nbatch + 1]
        .set(cu_seqlens)
        .at[_LANES - 1]
        .set(jnp.asarray(maxlen, jnp.int32))
    )
    padded = _padder(nbatch, _MAXLEN, total, d)(flat, cu_pad, zero_block)
    lengths = cu_seqlens[1:] - cu_seqlens[:-1]
    return padded, lengths


# speedup vs baseline: 11.9031x; 1.0006x over previous
import functools

import jax
import jax.numpy as jnp
from jax import lax
from jax.experimental import pallas as pl
from jax.experimental.pallas import tpu as pltpu
from jax.experimental.pallas import tpu_sc as plsc

_LANES = 16
_CHUNK = 128
_MAXLEN = 2048


@functools.lru_cache(maxsize=None)
def _padder(nbatch, maxlen, total, d):
    info = plsc.get_sparse_core_info()
    nc, ns = info.num_cores, info.num_subcores
    nw = nc * ns
    out_rows = nbatch * maxlen
    assert out_rows % nw == 0
    rpw = out_rows // nw
    assert maxlen % rpw == 0 and rpw % _CHUNK == 0
    wpb = maxlen // rpw
    nchunks = rpw // _CHUNK
    mesh = plsc.VectorSubcoreMesh(core_axis_name="c", subcore_axis_name="s")

    @functools.partial(
        pl.kernel,
        mesh=mesh,
        out_type=jax.ShapeDtypeStruct((nbatch, maxlen, d), jnp.float32),
        compiler_params=pltpu.CompilerParams(
            needs_layout_passes=False,
            skip_device_barrier=True,
            disable_bounds_checks=True,
            disable_semaphore_checks=True,
        ),
        scratch_types=[
            pltpu.VMEM((_LANES,), jnp.int32),
            pltpu.VMEM((_CHUNK,), jnp.int32),
            pltpu.VMEM((nchunks, _CHUNK, d), jnp.float32),
            pltpu.VMEM((_CHUNK, d), jnp.float32),
            pltpu.SemaphoreType.DMA((nchunks,)),
            pltpu.SemaphoreType.DMA,
            pltpu.SemaphoreType.DMA,
        ],
    )
    def k(flat_hbm, cu_hbm, zero_hbm, out_hbm,
          cu_v, idx_v, rows_v, zbuf, isem, zsem, osem):
        wid = lax.axis_index("s") * nc + lax.axis_index("c")
        b = wid // wpb
        t0 = pl.multiple_of((wid % wpb) * rpw, _CHUNK)
        zstage = pltpu.async_copy(zero_hbm, zbuf, zsem)
        pltpu.sync_copy(cu_hbm, cu_v)
        cu_vec = cu_v[...]
        iota = lax.iota(jnp.int32, _LANES)

        def lane_scalar(i):
            return jnp.sum(jnp.where(iota == i, cu_vec, 0))

        start = lane_scalar(b)
        length = jnp.minimum(lane_scalar(b + 1) - start, lane_scalar(_LANES - 1))
        nv = jnp.clip(length - t0, 0, rpw)
        aligned = start % 8 == 0
        fast_cs = [(nv >= (c + 1) * _CHUNK) & aligned for c in range(nchunks)]

        for c in range(nchunks):
            c0 = c * _CHUNK

            @pl.when(fast_cs[c])
            def _full():
                src = pl.multiple_of((start // 8) * 8 + t0 + c0, 8)
                pltpu.async_copy(
                    flat_hbm.at[pl.ds(src, _CHUNK)], rows_v.at[c], isem.at[c]
                )

            @pl.when(~fast_cs[c] & (nv > c0))
            def _gather():
                for j in range(_CHUNK // _LANES):
                    t = t0 + c0 + j * _LANES + iota
                    idx = jnp.clip(jnp.where(t < length, start + t, 0), 0, total - 1)
                    idx_v[pl.ds(j * _LANES, _LANES)] = idx
                pltpu.async_copy(flat_hbm.at[idx_v], rows_v.at[c], isem.at[c])

        zstage.wait()

        for c in range(nchunks):
            c0 = c * _CHUNK
            dst = out_hbm.at[b, pl.ds(t0 + c0, _CHUNK)]

            @pl.when(nv > c0)
            def _wait_in():
                pltpu.make_async_copy(
                    flat_hbm.at[pl.ds(0, _CHUNK)], rows_v.at[c], isem.at[c]
                ).wait()

            @pl.when(~fast_cs[c] & (nv > c0))
            def _mask_tail():
                def body(row, carry):
                    for g in range(d // _LANES):
                        rows_v[c, row, pl.ds(g * _LANES, _LANES)] = jnp.zeros(
                            (_LANES,), jnp.float32
                        )
                    return carry

                lax.fori_loop(jnp.maximum(nv - c0, 0), _CHUNK, body, 0)

            @pl.when(nv > c0)
            def _data():
                pltpu.async_copy(rows_v.at[c], dst, osem)

            @pl.when(nv <= c0)
            def _pad():
                pltpu.async_copy(zbuf, dst, osem)

        for c in range(nchunks):
            pltpu.make_async_copy(zero_hbm, zbuf, osem).wait()

    return k


def kernel(flat, cu_seqlens, maxlen):
    total, d = flat.shape
    nbatch = cu_seqlens.shape[0] - 1
    zero_block = jnp.zeros((_CHUNK, d), flat.dtype)
    cu_pad = (
        jnp.zeros((_LANES,), jnp.int32)
        .at[: nbatch + 1]
        .set(cu_seqlens)
        .at[_LANES - 1]
        .set(jnp.asarray(maxlen, jnp.int32))
    )
    padded = _padder(nbatch, _MAXLEN, total, d)(flat, cu_pad, zero_block)
    lengths = cu_seqlens[1:] - cu_seqlens[:-1]
    return padded, lengths


# speedup vs baseline: 11.9492x; 1.0039x over previous
import functools

import jax
import jax.numpy as jnp
from jax import lax
from jax.experimental import pallas as pl
from jax.experimental.pallas import tpu as pltpu
from jax.experimental.pallas import tpu_sc as plsc

_LANES = 16
_CHUNK = 128
_MAXLEN = 2048


@functools.lru_cache(maxsize=None)
def _padder(nbatch, maxlen, total, d):
    info = plsc.get_sparse_core_info()
    nc, ns = info.num_cores, info.num_subcores
    nw = nc * ns
    out_rows = nbatch * maxlen
    assert out_rows % nw == 0
    rpw = out_rows // nw
    assert maxlen % rpw == 0 and rpw % _CHUNK == 0
    wpb = maxlen // rpw
    nchunks = rpw // _CHUNK
    mesh = plsc.VectorSubcoreMesh(core_axis_name="c", subcore_axis_name="s")

    @functools.partial(
        pl.kernel,
        mesh=mesh,
        out_type=jax.ShapeDtypeStruct((nbatch, maxlen, d), jnp.float32),
        compiler_params=pltpu.CompilerParams(needs_layout_passes=False),
        scratch_types=[
            pltpu.VMEM((_LANES,), jnp.int32),
            pltpu.VMEM((_CHUNK,), jnp.int32),
            pltpu.VMEM((nchunks, _CHUNK, d), jnp.float32),
            pltpu.VMEM((_CHUNK, d), jnp.float32),
            pltpu.SemaphoreType.DMA((nchunks,)),
            pltpu.SemaphoreType.DMA,
            pltpu.SemaphoreType.DMA,
        ],
    )
    def k(flat_hbm, cu_hbm, zero_hbm, out_hbm,
          cu_v, idx_v, rows_v, zbuf, isem, zsem, osem):
        wid = lax.axis_index("s") * nc + lax.axis_index("c")
        b = wid // wpb
        t0 = pl.multiple_of((wid % wpb) * rpw, _CHUNK)
        zstage = pltpu.async_copy(zero_hbm, zbuf, zsem)
        pltpu.sync_copy(cu_hbm, cu_v)
        cu_vec = cu_v[...]
        iota = lax.iota(jnp.int32, _LANES)

        def lane_scalar(i):
            return jnp.sum(jnp.where(iota == i, cu_vec, 0))

        start = lane_scalar(b)
        length = jnp.minimum(lane_scalar(b + 1) - start, lane_scalar(_LANES - 1))
        nv = jnp.clip(length - t0, 0, rpw)
        aligned = start % 8 == 0
        fast_cs = [(nv >= (c + 1) * _CHUNK) & aligned for c in range(nchunks)]

        for c in range(nchunks):
            c0 = c * _CHUNK

            @pl.when(fast_cs[c])
            def _full():
                src = pl.multiple_of((start // 8) * 8 + t0 + c0, 8)
                pltpu.async_copy(
                    flat_hbm.at[pl.ds(src, _CHUNK)], rows_v.at[c], isem.at[c]
                )

            @pl.when(~fast_cs[c] & (nv > c0))
            def _gather():
                for j in range(_CHUNK // _LANES):
                    t = t0 + c0 + j * _LANES + iota
                    idx = jnp.clip(jnp.where(t < length, start + t, 0), 0, total - 1)
                    idx_v[pl.ds(j * _LANES, _LANES)] = idx
                pltpu.async_copy(flat_hbm.at[idx_v], rows_v.at[c], isem.at[c])

        zstage.wait()

        for c in range(nchunks):
            c0 = c * _CHUNK
            dst = out_hbm.at[b, pl.ds(t0 + c0, _CHUNK)]

            @pl.when(nv > c0)
            def _wait_in():
                pltpu.make_async_copy(
                    flat_hbm.at[pl.ds(0, _CHUNK)], rows_v.at[c], isem.at[c]
                ).wait()

            @pl.when(~fast_cs[c] & (nv > c0))
            def _mask_tail():
                def body(row, carry):
                    for g in range(d // _LANES):
                        rows_v[c, row, pl.ds(g * _LANES, _LANES)] = jnp.zeros(
                            (_LANES,), jnp.float32
                        )
                    return carry

                lax.fori_loop(jnp.maximum(nv - c0, 0), _CHUNK, body, 0)

            @pl.when(nv > c0)
            def _data():
                pltpu.async_copy(rows_v.at[c], dst, osem)

            @pl.when(nv <= c0)
            def _pad():
                pltpu.async_copy(zbuf, dst, osem)

        for c in range(nchunks):
            pltpu.make_async_copy(zero_hbm, zbuf, osem).wait()

    return k


def kernel(flat, cu_seqlens, maxlen):
    total, d = flat.shape
    nbatch = cu_seqlens.shape[0] - 1
    zero_block = jnp.zeros((_CHUNK, d), flat.dtype)
    cu_pad = (
        jnp.zeros((_LANES,), jnp.int32)
        .at[: nbatch + 1]
        .set(cu_seqlens)
        .at[_LANES - 1]
        .set(jnp.asarray(maxlen, jnp.int32))
    )
    padded = _padder(nbatch, _MAXLEN, total, d)(flat, cu_pad, zero_block)
    lengths = cu_seqlens[1:] - cu_seqlens[:-1]
    return padded, lengths


# speedup vs baseline: 12.3523x; 1.0337x over previous
import functools

import jax
import jax.numpy as jnp
import numpy as np
from jax import lax
from jax.experimental import pallas as pl
from jax.experimental.pallas import tpu as pltpu
from jax.experimental.pallas import tpu_sc as plsc

_LANES = 16
_CHUNK = 128
_MAXLEN = 2048


@functools.lru_cache(maxsize=None)
def _padder(nbatch, maxlen, total, d):
    info = plsc.get_sparse_core_info()
    nc, ns = info.num_cores, info.num_subcores
    nw = nc * ns
    out_rows = nbatch * maxlen
    assert out_rows % nw == 0
    rpw = out_rows // nw
    assert maxlen % rpw == 0 and rpw % _CHUNK == 0
    wpb = maxlen // rpw
    nchunks = rpw // _CHUNK
    mesh = plsc.VectorSubcoreMesh(core_axis_name="c", subcore_axis_name="s")

    @functools.partial(
        pl.kernel,
        mesh=mesh,
        out_type=jax.ShapeDtypeStruct((nbatch, maxlen, d), jnp.float32),
        compiler_params=pltpu.CompilerParams(needs_layout_passes=False),
        scratch_types=[
            pltpu.VMEM((_LANES,), jnp.int32),
            pltpu.VMEM((_CHUNK,), jnp.int32),
            pltpu.VMEM((nchunks, _CHUNK, d), jnp.float32),
            pltpu.VMEM((_CHUNK, d), jnp.float32),
            pltpu.SemaphoreType.DMA((nchunks,)),
            pltpu.SemaphoreType.DMA,
            pltpu.SemaphoreType.DMA,
        ],
    )
    def k(flat_hbm, cu_hbm, zero_hbm, out_hbm,
          cu_v, idx_v, rows_v, zbuf, isem, zsem, osem):
        wid = lax.axis_index("s") * nc + lax.axis_index("c")
        b = wid // wpb
        t0 = pl.multiple_of((wid % wpb) * rpw, _CHUNK)
        pltpu.sync_copy(cu_hbm, cu_v)
        cu_vec = cu_v[...]
        iota = lax.iota(jnp.int32, _LANES)

        def lane_scalar(i):
            return jnp.sum(jnp.where(iota == i, cu_vec, 0))

        start = lane_scalar(b)
        length = jnp.minimum(lane_scalar(b + 1) - start, lane_scalar(_LANES - 1))
        nv = jnp.clip(length - t0, 0, rpw)
        aligned = start % 8 == 0
        fast_cs = [(nv >= (c + 1) * _CHUNK) & aligned for c in range(nchunks)]
        has_pad = nv < rpw

        @pl.when(has_pad)
        def _zstage():
            pltpu.async_copy(zero_hbm, zbuf, zsem)

        for c in range(nchunks):
            c0 = c * _CHUNK

            @pl.when(fast_cs[c])
            def _full():
                src = pl.multiple_of((start // 8) * 8 + t0 + c0, 8)
                pltpu.async_copy(
                    flat_hbm.at[pl.ds(src, _CHUNK)], rows_v.at[c], isem.at[c]
                )

            @pl.when(~fast_cs[c] & (nv > c0))
            def _gather():
                for j in range(_CHUNK // _LANES):
                    t = t0 + c0 + j * _LANES + iota
                    idx = jnp.clip(jnp.where(t < length, start + t, 0), 0, total - 1)
                    idx_v[pl.ds(j * _LANES, _LANES)] = idx
                pltpu.async_copy(flat_hbm.at[idx_v], rows_v.at[c], isem.at[c])

        @pl.when(has_pad)
        def _zwait():
            pltpu.make_async_copy(zero_hbm, zbuf, zsem).wait()

        for c in range(nchunks):
            c0 = c * _CHUNK
            dst = out_hbm.at[b, pl.ds(t0 + c0, _CHUNK)]

            @pl.when(nv > c0)
            def _wait_in():
                pltpu.make_async_copy(
                    flat_hbm.at[pl.ds(0, _CHUNK)], rows_v.at[c], isem.at[c]
                ).wait()

            @pl.when(~fast_cs[c] & (nv > c0))
            def _mask_tail():
                def body(row, carry):
                    for g in range(d // _LANES):
                        rows_v[c, row, pl.ds(g * _LANES, _LANES)] = jnp.zeros(
                            (_LANES,), jnp.float32
                        )
                    return carry

                lax.fori_loop(jnp.maximum(nv - c0, 0), _CHUNK, body, 0)

            @pl.when(nv > c0)
            def _data():
                pltpu.async_copy(rows_v.at[c], dst, osem)

            @pl.when(nv <= c0)
            def _pad():
                pltpu.async_copy(zbuf, dst, osem)

        for c in range(nchunks):
            pltpu.make_async_copy(zero_hbm, zbuf, osem).wait()

    return k


def kernel(flat, cu_seqlens, maxlen):
    total, d = flat.shape
    nbatch = cu_seqlens.shape[0] - 1
    zero_block = np.zeros((_CHUNK, d), np.float32)
    cu_pad = (
        jnp.zeros((_LANES,), jnp.int32)
        .at[: nbatch + 1]
        .set(cu_seqlens)
        .at[_LANES - 1]
        .set(jnp.asarray(maxlen, jnp.int32))
    )
    padded = _padder(nbatch, _MAXLEN, total, d)(flat, cu_pad, zero_block)
    lengths = cu_seqlens[1:] - cu_seqlens[:-1]
    return padded, lengths


# speedup vs baseline: 12.6063x; 1.0206x over previous
import functools

import jax
import jax.numpy as jnp
import numpy as np
from jax import lax
from jax.experimental import pallas as pl
from jax.experimental.pallas import tpu as pltpu
from jax.experimental.pallas import tpu_sc as plsc

_LANES = 16
_CHUNK = 128
_MAXLEN = 2048


@functools.lru_cache(maxsize=None)
def _padder(nbatch, maxlen, total, d):
    info = plsc.get_sparse_core_info()
    nc, ns = info.num_cores, info.num_subcores
    nw = nc * ns
    out_rows = nbatch * maxlen
    assert out_rows % nw == 0
    rpw = out_rows // nw
    assert maxlen % rpw == 0 and rpw % _CHUNK == 0
    wpb = maxlen // rpw
    nchunks = rpw // _CHUNK
    mesh = plsc.VectorSubcoreMesh(core_axis_name="c", subcore_axis_name="s")

    @functools.partial(
        pl.kernel,
        mesh=mesh,
        out_type=jax.ShapeDtypeStruct((nbatch, maxlen, d), jnp.float32),
        compiler_params=pltpu.CompilerParams(needs_layout_passes=False),
        scratch_types=[
            pltpu.VMEM((_LANES,), jnp.int32),
            pltpu.VMEM((_CHUNK,), jnp.int32),
            pltpu.VMEM((nchunks, _CHUNK, d), jnp.float32),
            pltpu.VMEM((_CHUNK, d), jnp.float32),
            pltpu.SemaphoreType.DMA((nchunks,)),
            pltpu.SemaphoreType.DMA,
            pltpu.SemaphoreType.DMA,
        ],
    )
    def k(flat_hbm, cu_hbm, zero_hbm, out_hbm,
          cu_v, idx_v, rows_v, zbuf, isem, zsem, osem):
        wid = lax.axis_index("s") * nc + lax.axis_index("c")
        b = wid // wpb
        t0 = pl.multiple_of((wid % wpb) * rpw, _CHUNK)
        pltpu.sync_copy(cu_hbm, cu_v)
        cu_vec = cu_v[...]
        iota = lax.iota(jnp.int32, _LANES)

        def lane_scalar(i):
            return jnp.sum(jnp.where(iota == i, cu_vec, 0))

        start = lane_scalar(b)
        length = lane_scalar(b + 1) - start
        nv = jnp.clip(length - t0, 0, rpw)
        aligned = start % 8 == 0
        fast_cs = [(nv >= (c + 1) * _CHUNK) & aligned for c in range(nchunks)]
        has_pad = nv < rpw

        @pl.when(has_pad)
        def _zstage():
            pltpu.async_copy(zero_hbm, zbuf, zsem)

        for c in range(nchunks):
            c0 = c * _CHUNK

            @pl.when(fast_cs[c])
            def _full():
                src = pl.multiple_of((start // 8) * 8 + t0 + c0, 8)
                pltpu.async_copy(
                    flat_hbm.at[pl.ds(src, _CHUNK)], rows_v.at[c], isem.at[c]
                )

            @pl.when(~fast_cs[c] & (nv > c0))
            def _gather():
                for j in range(_CHUNK // _LANES):
                    t = t0 + c0 + j * _LANES + iota
                    idx = jnp.clip(jnp.where(t < length, start + t, 0), 0, total - 1)
                    idx_v[pl.ds(j * _LANES, _LANES)] = idx
                pltpu.async_copy(flat_hbm.at[idx_v], rows_v.at[c], isem.at[c])

        @pl.when(has_pad)
        def _zwait():
            pltpu.make_async_copy(zero_hbm, zbuf, zsem).wait()

        for c in range(nchunks):
            c0 = c * _CHUNK
            dst = out_hbm.at[b, pl.ds(t0 + c0, _CHUNK)]

            @pl.when(nv > c0)
            def _wait_in():
                pltpu.make_async_copy(
                    flat_hbm.at[pl.ds(0, _CHUNK)], rows_v.at[c], isem.at[c]
                ).wait()

            @pl.when(~fast_cs[c] & (nv > c0))
            def _mask_tail():
                def body(row, carry):
                    for g in range(d // _LANES):
                        rows_v[c, row, pl.ds(g * _LANES, _LANES)] = jnp.zeros(
                            (_LANES,), jnp.float32
                        )
                    return carry

                lax.fori_loop(jnp.maximum(nv - c0, 0), _CHUNK, body, 0)

            @pl.when(nv > c0)
            def _data():
                pltpu.async_copy(rows_v.at[c], dst, osem)

            @pl.when(nv <= c0)
            def _pad():
                pltpu.async_copy(zbuf, dst, osem)

        for c in range(nchunks):
            pltpu.make_async_copy(zero_hbm, zbuf, osem).wait()

    return k


def kernel(flat, cu_seqlens, maxlen):
    total, d = flat.shape
    nbatch = cu_seqlens.shape[0] - 1
    del maxlen
    zero_block = np.zeros((_CHUNK, d), np.float32)
    cu_pad = jnp.zeros((_LANES,), jnp.int32).at[: nbatch + 1].set(cu_seqlens)
    padded = _padder(nbatch, _MAXLEN, total, d)(flat, cu_pad, zero_block)
    lengths = cu_seqlens[1:] - cu_seqlens[:-1]
    return padded, lengths


# speedup vs baseline: 12.6994x; 1.0074x over previous
import functools

import jax
import jax.numpy as jnp
import numpy as np
from jax import lax
from jax.experimental import pallas as pl
from jax.experimental.pallas import tpu as pltpu
from jax.experimental.pallas import tpu_sc as plsc

_LANES = 16
_CHUNK = 128
_MAXLEN = 2048


@functools.lru_cache(maxsize=None)
def _padder(nbatch, maxlen, total, d):
    info = plsc.get_sparse_core_info()
    nc, ns = info.num_cores, info.num_subcores
    nw = nc * ns
    out_rows = nbatch * maxlen
    assert out_rows % nw == 0
    rpw = out_rows // nw
    assert maxlen % rpw == 0 and rpw % _CHUNK == 0
    wpb = maxlen // rpw
    nchunks = rpw // _CHUNK
    mesh = plsc.VectorSubcoreMesh(core_axis_name="c", subcore_axis_name="s")

    @functools.partial(
        pl.kernel,
        mesh=mesh,
        out_type=jax.ShapeDtypeStruct((nbatch, maxlen, d), jnp.float32),
        compiler_params=pltpu.CompilerParams(needs_layout_passes=False),
        scratch_types=[
            pltpu.VMEM((_LANES,), jnp.int32),
            pltpu.VMEM((nchunks, _CHUNK), jnp.int32),
            pltpu.VMEM((nchunks, _CHUNK, d), jnp.float32),
            pltpu.VMEM((_CHUNK, d), jnp.float32),
            pltpu.SemaphoreType.DMA((nchunks,)),
            pltpu.SemaphoreType.DMA,
            pltpu.SemaphoreType.DMA,
        ],
    )
    def k(flat_hbm, cu_hbm, zero_hbm, out_hbm,
          cu_v, idx_v, rows_v, zbuf, isem, zsem, osem):
        wid = lax.axis_index("s") * nc + lax.axis_index("c")
        b = wid // wpb
        t0 = pl.multiple_of((wid % wpb) * rpw, _CHUNK)
        pltpu.sync_copy(cu_hbm, cu_v)
        cu_vec = cu_v[...]
        iota = lax.iota(jnp.int32, _LANES)

        def lane_scalar(i):
            return jnp.sum(jnp.where(iota == i, cu_vec, 0))

        start = lane_scalar(b)
        length = lane_scalar(b + 1) - start
        nv = jnp.clip(length - t0, 0, rpw)
        aligned = start % 8 == 0
        fast_cs = [(nv >= (c + 1) * _CHUNK) & aligned for c in range(nchunks)]
        has_pad = nv < rpw

        @pl.when(has_pad)
        def _zstage():
            pltpu.async_copy(zero_hbm, zbuf, zsem)

        for c in range(nchunks):
            c0 = c * _CHUNK

            @pl.when(fast_cs[c])
            def _full():
                src = pl.multiple_of((start // 8) * 8 + t0 + c0, 8)
                pltpu.async_copy(
                    flat_hbm.at[pl.ds(src, _CHUNK)], rows_v.at[c], isem.at[c]
                )

            @pl.when(~fast_cs[c] & (nv > c0))
            def _gather():
                for j in range(_CHUNK // _LANES):
                    t = t0 + c0 + j * _LANES + iota
                    idx = jnp.clip(jnp.where(t < length, start + t, 0), 0, total - 1)
                    idx_v[c, pl.ds(j * _LANES, _LANES)] = idx
                pltpu.async_copy(flat_hbm.at[idx_v.at[c]], rows_v.at[c], isem.at[c])

        @pl.when(has_pad)
        def _zwait():
            pltpu.make_async_copy(zero_hbm, zbuf, zsem).wait()

        for c in range(nchunks):
            c0 = c * _CHUNK
            dst = out_hbm.at[b, pl.ds(t0 + c0, _CHUNK)]

            @pl.when(nv > c0)
            def _wait_in():
                pltpu.make_async_copy(
                    flat_hbm.at[pl.ds(0, _CHUNK)], rows_v.at[c], isem.at[c]
                ).wait()

            @pl.when(~fast_cs[c] & (nv > c0))
            def _mask_tail():
                def body(row, carry):
                    for g in range(d // _LANES):
                        rows_v[c, row, pl.ds(g * _LANES, _LANES)] = jnp.zeros(
                            (_LANES,), jnp.float32
                        )
                    return carry

                lax.fori_loop(jnp.maximum(nv - c0, 0), _CHUNK, body, 0)

            @pl.when(nv > c0)
            def _data():
                pltpu.async_copy(rows_v.at[c], dst, osem)

            @pl.when(nv <= c0)
            def _pad():
                pltpu.async_copy(zbuf, dst, osem)

        for c in range(nchunks):
            pltpu.make_async_copy(zero_hbm, zbuf, osem).wait()

    return k


def kernel(flat, cu_seqlens, maxlen):
    total, d = flat.shape
    nbatch = cu_seqlens.shape[0] - 1
    del maxlen
    zero_block = np.zeros((_CHUNK, d), np.float32)
    cu_pad = jnp.zeros((_LANES,), jnp.int32).at[: nbatch + 1].set(cu_seqlens)
    padded = _padder(nbatch, _MAXLEN, total, d)(flat, cu_pad, zero_block)
    lengths = cu_seqlens[1:] - cu_seqlens[:-1]
    return padded, lengths
